# Initial kernel scaffold; baseline (speedup 1.0000x reference)
import jax
import jax.numpy as jnp
from jax.experimental import pallas as pl


def kernel(input_feature, adj, W1, a1, W_out, a_out):
    raise NotImplementedError("write your pallas kernel here")


# speedup vs baseline: 1.5838x; 1.5838x over previous
import functools

import jax
import jax.numpy as jnp
from jax.experimental import pallas as pl
from jax.experimental.pallas import tpu as pltpu

_MASK_VAL = -9e15


def _gat_layer_kernel(x_ref, adj_ref, w_ref, a_ref, out_ref, wh_scr, *,
                      rb_size, elu, relu):
    rb = pl.program_id(1)
    heads, _, f_out = w_ref.shape
    n = x_ref.shape[1]

    @pl.when(rb == 0)
    def _project():
        x = x_ref[0]
        for h in range(heads):
            wh_scr[h] = jnp.dot(x, w_ref[h], preferred_element_type=jnp.float32)

    adj = adj_ref[0] > 0
    row0 = rb * rb_size
    acc = jnp.zeros((rb_size, f_out), jnp.float32)
    for h in range(heads):
        wh = wh_scr[h]
        wh_rows = wh_scr[h, pl.ds(row0, rb_size), :]
        a_src = a_ref[h, :, :f_out]
        a_dst = a_ref[h, :, f_out:]
        e_src = jax.lax.dot_general(
            wh_rows, a_src, (((1,), (1,)), ((), ())),
            preferred_element_type=jnp.float32)
        e_dst = jax.lax.dot_general(
            a_dst, wh, (((1,), (1,)), ((), ())),
            preferred_element_type=jnp.float32)
        e = e_src + e_dst
        e = jnp.where(e >= 0, e, 0.2 * e)
        e = jnp.where(adj, e, jnp.float32(_MASK_VAL))
        m = jnp.max(e, axis=1, keepdims=True)
        p = jnp.exp(e - m)
        att = p / jnp.sum(p, axis=1, keepdims=True)
        hp = jnp.dot(att, wh, preferred_element_type=jnp.float32)
        if elu:
            hp = jnp.where(hp > 0, hp, jnp.exp(hp) - 1.0)
        acc = acc + hp
    if heads > 1:
        acc = acc * (1.0 / heads)
    if relu:
        acc = jnp.maximum(acc, 0.0)
    out_ref[0] = acc


def _gat_layer(x, adj, w, a_t, *, elu, relu, rb_size=256):
    b, n, f_in = x.shape
    heads, _, f_out = w.shape
    kern = functools.partial(_gat_layer_kernel, rb_size=rb_size, elu=elu,
                             relu=relu)
    return pl.pallas_call(
        kern,
        grid=(b, n // rb_size),
        in_specs=[
            pl.BlockSpec((1, n, f_in), lambda i, r: (i, 0, 0)),
            pl.BlockSpec((1, rb_size, n), lambda i, r: (i, r, 0)),
            pl.BlockSpec((heads, f_in, f_out), lambda i, r: (0, 0, 0)),
            pl.BlockSpec((heads, 1, 2 * f_out), lambda i, r: (0, 0, 0)),
        ],
        out_specs=pl.BlockSpec((1, rb_size, f_out), lambda i, r: (i, r, 0)),
        out_shape=jax.ShapeDtypeStruct((b, n, f_out), jnp.float32),
        scratch_shapes=[pltpu.VMEM((heads, n, f_out), jnp.float32)],
        compiler_params=pltpu.CompilerParams(
            dimension_semantics=("arbitrary", "arbitrary")),
    )(x, adj, w, a_t)


def kernel(input_feature, adj, W1, a1, W_out, a_out):
    a1_t = jnp.transpose(a1, (0, 2, 1))
    x = _gat_layer(input_feature, adj, W1, a1_t, elu=True, relu=False)
    w2 = W_out[None]
    a2_t = jnp.transpose(a_out, (1, 0))[None]
    return _gat_layer(x, adj, w2, a2_t, elu=False, relu=True)


# speedup vs baseline: 1.8421x; 1.1631x over previous
import functools

import jax
import jax.numpy as jnp
from jax.experimental import pallas as pl
from jax.experimental.pallas import tpu as pltpu

_MASK_VAL = -9e15


def _gat_layer_kernel(x_ref, adj_ref, w_ref, a_ref, out_ref, wh_scr, es_scr,
                      ed_scr, *, rb_size, elu, relu):
    rb = pl.program_id(1)
    heads, _, f_out = w_ref.shape

    @pl.when(rb == 0)
    def _project():
        x = x_ref[0]
        for h in range(heads):
            wh = jnp.dot(x, w_ref[h], preferred_element_type=jnp.float32)
            wh_scr[h] = wh
            es_scr[h] = jax.lax.dot_general(
                wh, a_ref[h, :, :f_out], (((1,), (1,)), ((), ())),
                preferred_element_type=jnp.float32)
            ed_scr[h] = jax.lax.dot_general(
                a_ref[h, :, f_out:], wh, (((1,), (1,)), ((), ())),
                preferred_element_type=jnp.float32)

    madd = (adj_ref[0].astype(jnp.float32) - 1.0) * (-_MASK_VAL)
    row0 = rb * rb_size
    acc = jnp.zeros((rb_size, f_out), jnp.float32)
    for h in range(heads):
        e = es_scr[h, pl.ds(row0, rb_size), :] + ed_scr[h]
        e = jnp.maximum(e, 0.2 * e) + madd
        m = jnp.max(e, axis=1, keepdims=True)
        p = jnp.exp(e - m)
        s = jnp.sum(p, axis=1, keepdims=True)
        hp = jnp.dot(p, wh_scr[h], preferred_element_type=jnp.float32) / s
        if elu:
            hp = jnp.where(hp > 0, hp, jnp.exp(hp) - 1.0)
        acc = acc + hp
    if heads > 1:
        acc = acc * (1.0 / heads)
    if relu:
        acc = jnp.maximum(acc, 0.0)
    out_ref[0] = acc


def _gat_layer(x, adj, w, a_t, *, elu, relu, rb_size=256):
    b, n, f_in = x.shape
    heads, _, f_out = w.shape
    kern = functools.partial(_gat_layer_kernel, rb_size=rb_size, elu=elu,
                             relu=relu)
    return pl.pallas_call(
        kern,
        grid=(b, n // rb_size),
        in_specs=[
            pl.BlockSpec((1, n, f_in), lambda i, r: (i, 0, 0)),
            pl.BlockSpec((1, rb_size, n), lambda i, r: (i, r, 0)),
            pl.BlockSpec((heads, f_in, f_out), lambda i, r: (0, 0, 0)),
            pl.BlockSpec((heads, 1, 2 * f_out), lambda i, r: (0, 0, 0)),
        ],
        out_specs=pl.BlockSpec((1, rb_size, f_out), lambda i, r: (i, r, 0)),
        out_shape=jax.ShapeDtypeStruct((b, n, f_out), jnp.float32),
        scratch_shapes=[
            pltpu.VMEM((heads, n, f_out), jnp.float32),
            pltpu.VMEM((heads, n, 1), jnp.float32),
            pltpu.VMEM((heads, 1, n), jnp.float32),
        ],
        compiler_params=pltpu.CompilerParams(
            dimension_semantics=("parallel", "arbitrary")),
    )(x, adj, w, a_t)


def kernel(input_feature, adj, W1, a1, W_out, a_out):
    a1_t = jnp.transpose(a1, (0, 2, 1))
    x = _gat_layer(input_feature, adj, W1, a1_t, elu=True, relu=False)
    w2 = W_out[None]
    a2_t = jnp.transpose(a_out, (1, 0))[None]
    return _gat_layer(x, adj, w2, a2_t, elu=False, relu=True)


# speedup vs baseline: 2.3138x; 1.2560x over previous
import functools

import jax
import jax.numpy as jnp
from jax.experimental import pallas as pl
from jax.experimental.pallas import tpu as pltpu

_MASK_VAL = -9e15
_LOG2E = 1.4426950408889634


def _gat_layer_kernel(x_ref, adj_ref, w_ref, a_ref, out_ref, wh_scr, es_scr,
                      ed_scr, *, rb_size, elu, relu):
    rb = pl.program_id(1)
    heads, _, f_out = w_ref.shape
    n = x_ref.shape[1]

    @pl.when(rb == 0)
    def _project():
        x = x_ref[0]
        col = jax.lax.broadcasted_iota(jnp.int32, (n, f_out), 1)
        ones_first_col = jnp.where(col == 0, 1.0, 0.0).astype(jnp.float32)
        for h in range(heads):
            wh = jnp.dot(x, w_ref[h], preferred_element_type=jnp.float32)
            wh_scr[h, :, :f_out] = wh
            wh_scr[h, :, f_out:] = ones_first_col
            es_scr[h] = jax.lax.dot_general(
                wh, a_ref[h, :, :f_out], (((1,), (1,)), ((), ())),
                preferred_element_type=jnp.float32)
            ed_scr[h] = jax.lax.dot_general(
                a_ref[h, :, f_out:], wh, (((1,), (1,)), ((), ())),
                preferred_element_type=jnp.float32)

    madd = (adj_ref[0].astype(jnp.float32) - 1.0) * (-_MASK_VAL)
    row0 = rb * rb_size
    acc = jnp.zeros((rb_size, f_out), jnp.float32)
    for h in range(heads):
        e = es_scr[h, pl.ds(row0, rb_size), :] + ed_scr[h]
        e = jnp.maximum(e, 0.2 * e) + madd
        m = jnp.max(e, axis=1, keepdims=True)
        p = jnp.exp2(e - m)
        hp_aug = jnp.dot(p, wh_scr[h],
                         preferred_element_type=jnp.float32)
        hp = hp_aug[:, :f_out] / hp_aug[:, f_out:f_out + 1]
        if elu:
            hp = jnp.where(hp > 0, hp, jnp.exp(hp) - 1.0)
        acc = acc + hp
    if heads > 1:
        acc = acc * (1.0 / heads)
    if relu:
        acc = jnp.maximum(acc, 0.0)
    out_ref[0] = acc


def _gat_layer(x, adj, w, a_t, *, elu, relu, rb_size=1024):
    b, n, f_in = x.shape
    heads, _, f_out = w.shape
    kern = functools.partial(_gat_layer_kernel, rb_size=rb_size, elu=elu,
                             relu=relu)
    return pl.pallas_call(
        kern,
        grid=(b, n // rb_size),
        in_specs=[
            pl.BlockSpec((1, n, f_in), lambda i, r: (i, 0, 0)),
            pl.BlockSpec((1, rb_size, n), lambda i, r: (i, r, 0)),
            pl.BlockSpec((heads, f_in, f_out), lambda i, r: (0, 0, 0)),
            pl.BlockSpec((heads, 1, 2 * f_out), lambda i, r: (0, 0, 0)),
        ],
        out_specs=pl.BlockSpec((1, rb_size, f_out), lambda i, r: (i, r, 0)),
        out_shape=jax.ShapeDtypeStruct((b, n, f_out), jnp.float32),
        scratch_shapes=[
            pltpu.VMEM((heads, n, 2 * f_out), jnp.float32),
            pltpu.VMEM((heads, n, 1), jnp.float32),
            pltpu.VMEM((heads, 1, n), jnp.float32),
        ],
        compiler_params=pltpu.CompilerParams(
            dimension_semantics=("parallel", "arbitrary")),
    )(x, adj, w, a_t)


def kernel(input_feature, adj, W1, a1, W_out, a_out):
    a1_t = jnp.transpose(a1, (0, 2, 1)) * _LOG2E
    x = _gat_layer(input_feature, adj, W1, a1_t, elu=True, relu=False)
    w2 = W_out[None]
    a2_t = jnp.transpose(a_out, (1, 0))[None] * _LOG2E
    return _gat_layer(x, adj, w2, a2_t, elu=False, relu=True)


# speedup vs baseline: 2.4133x; 1.0430x over previous
import jax
import jax.numpy as jnp
from jax.experimental import pallas as pl
from jax.experimental.pallas import tpu as pltpu

_MASK_VAL = -9e15
_LOG2E = 1.4426950408889634
_HEADS = 4


def _attention_pass(wh_scr, es_scr, ed_scr, madd_scr, h, n, f_out):
    e = es_scr[h] + ed_scr[h]
    e = jnp.maximum(e, jnp.bfloat16(0.2) * e) + madd_scr[...]
    m = jnp.max(e, axis=1, keepdims=True)
    p = jnp.exp2(e - m)
    return jnp.dot(p, wh_scr[h], preferred_element_type=jnp.float32)


def _project(x, w, a_src, a_dst, wh_scr, es_scr, ed_scr, h, n, f_out):
    wh = jnp.dot(x, w, preferred_element_type=jnp.float32)
    wh_scr[h, :, :f_out] = wh.astype(jnp.bfloat16)
    col = jax.lax.broadcasted_iota(jnp.int32, (n, f_out), 1)
    wh_scr[h, :, f_out:] = jnp.where(col == 0, 1.0, 0.0).astype(jnp.bfloat16)
    es = jax.lax.dot_general(wh, a_src, (((1,), (1,)), ((), ())),
                             preferred_element_type=jnp.float32)
    es_scr[h] = es.astype(jnp.bfloat16)
    ed = jax.lax.dot_general(a_dst, wh, (((1,), (1,)), ((), ())),
                             preferred_element_type=jnp.float32)
    ed_scr[h] = ed.astype(jnp.bfloat16)


def _gat_kernel(x_ref, adj_ref, w1_ref, a1_ref, w2_ref, a2_ref, out_ref,
                wh_scr, es_scr, ed_scr, madd_scr, x_scr):
    layer = pl.program_id(1)
    n = x_ref.shape[1]
    f_out = w2_ref.shape[-1]

    @pl.when(layer == 0)
    def _hidden_layer():
        madd_scr[...] = ((adj_ref[0].astype(jnp.bfloat16) - jnp.bfloat16(1.0))
                         * jnp.bfloat16(-_MASK_VAL))
        x = x_ref[0]
        for h in range(_HEADS):
            _project(x, w1_ref[h], a1_ref[h, :, :f_out], a1_ref[h, :, f_out:],
                     wh_scr, es_scr, ed_scr, h, n, f_out)
        acc = jnp.zeros((n, f_out), jnp.float32)
        for h in range(_HEADS):
            hp_aug = _attention_pass(wh_scr, es_scr, ed_scr, madd_scr,
                                     h, n, f_out)
            hp = hp_aug[:, :f_out] / hp_aug[:, f_out:f_out + 1]
            hp = jnp.where(hp > 0, hp, jnp.exp(hp) - 1.0)
            acc = acc + hp
        x_scr[...] = acc * (1.0 / _HEADS)

    @pl.when(layer == 1)
    def _output_layer():
        _project(x_scr[...], w2_ref[0], a2_ref[0, :, :f_out],
                 a2_ref[0, :, f_out:], wh_scr, es_scr, ed_scr, 0, n, f_out)
        hp_aug = _attention_pass(wh_scr, es_scr, ed_scr, madd_scr,
                                 0, n, f_out)
        hp = hp_aug[:, :f_out] / hp_aug[:, f_out:f_out + 1]
        out_ref[0] = jnp.maximum(hp, 0.0)


def kernel(input_feature, adj, W1, a1, W_out, a_out):
    b, n, f_in = input_feature.shape
    heads, _, f_out = W1.shape
    a1_t = jnp.transpose(a1, (0, 2, 1)) * _LOG2E
    w2 = W_out[None]
    a2_t = jnp.transpose(a_out, (1, 0))[None] * _LOG2E
    return pl.pallas_call(
        _gat_kernel,
        grid=(b, 2),
        in_specs=[
            pl.BlockSpec((1, n, f_in), lambda i, l: (i, 0, 0)),
            pl.BlockSpec((1, n, n), lambda i, l: (i, 0, 0)),
            pl.BlockSpec((heads, f_in, f_out), lambda i, l: (0, 0, 0)),
            pl.BlockSpec((heads, 1, 2 * f_out), lambda i, l: (0, 0, 0)),
            pl.BlockSpec((1, f_out, f_out), lambda i, l: (0, 0, 0)),
            pl.BlockSpec((1, 1, 2 * f_out), lambda i, l: (0, 0, 0)),
        ],
        out_specs=pl.BlockSpec((1, n, f_out), lambda i, l: (i, 0, 0)),
        out_shape=jax.ShapeDtypeStruct((b, n, f_out), jnp.float32),
        scratch_shapes=[
            pltpu.VMEM((heads, n, 2 * f_out), jnp.bfloat16),
            pltpu.VMEM((heads, n, 1), jnp.bfloat16),
            pltpu.VMEM((heads, 1, n), jnp.bfloat16),
            pltpu.VMEM((n, n), jnp.bfloat16),
            pltpu.VMEM((n, f_out), jnp.float32),
        ],
        compiler_params=pltpu.CompilerParams(
            dimension_semantics=("parallel", "arbitrary")),
    )(input_feature, adj, W1, a1_t, w2, a2_t)
